# baseline (speedup 1.0000x reference)
import functools
import math

import jax
import jax.numpy as jnp
from jax import lax
from jax.experimental import pallas as pl
from jax.experimental.pallas import tpu as pltpu
from jax.experimental.pallas import tpu_sc as plsc

N = 100000
COV = 128
T = 4
OUT = 5
H = 32
E = 200000

NC = 2
NS = 16
NW = NC * NS

ACC_ROWS = 100096
_SUB_ROWS = ACC_ROWS // NS
_ZROWS = 256

_BLK = 2000


def _pad_plan(num, k):
    nc = max(1, math.ceil(num / (NW * k)))
    return nc, k, NW * nc * k


def _sc_mesh():
    return plsc.VectorSubcoreMesh(core_axis_name="c", subcore_axis_name="s")


_SC_PARAMS = pltpu.CompilerParams(use_tc_tiling_on_sc=False)


def _fill_zeros(zbuf):
    @pl.loop(0, zbuf.shape[0])
    def _(i):
        zbuf[i, :] = jnp.zeros((16,), jnp.float32)


def _zero_acc(acc, zbuf, sid, sem):
    base = sid * _SUB_ROWS
    zr = zbuf.shape[0] // 2
    hs = []
    off = 0
    while off < _SUB_ROWS:
        rows = min(zr, _SUB_ROWS - off)
        hs.append(pltpu.async_copy(
            zbuf.at[pl.ds(0, rows)], acc.at[pl.ds(base + off, rows)], sem))
        if len(hs) >= 5:
            hs[-5].wait()
        off += rows
    for h in hs[-4:]:
        h.wait()


def _acc_to_out(acc, zbuf, out_hbm, cid, sid, sem_r, sem_w):
    base = sid * _SUB_ROWS
    zr = zbuf.shape[0] // 2
    wh = {}
    j = 0
    off = 0
    while off < _SUB_ROWS:
        rows = min(zr, _SUB_ROWS - off)
        b = j & 1
        if j >= 2:
            wh[j - 2].wait()
        pltpu.async_copy(acc.at[pl.ds(base + off, rows)],
                         zbuf.at[pl.ds(b * zr, rows)], sem_r[b]).wait()
        wh[j] = pltpu.async_copy(zbuf.at[pl.ds(b * zr, rows)],
                                 out_hbm.at[cid, pl.ds(base + off, rows)],
                                 sem_w[b])
        j += 1
        off += rows
    for jj in range(max(0, j - 2), j):
        wh[jj].wait()


@functools.cache
def _sc_count(nc, k):
    @functools.partial(
        pl.kernel,
        out_type=jax.ShapeDtypeStruct((NC, ACC_ROWS, 16), jnp.float32),
        mesh=_sc_mesh(),
        compiler_params=_SC_PARAMS,
        scratch_types=[
            pltpu.VMEM((3, k), jnp.int32),
            pltpu.VMEM((k, 16), jnp.float32),
            pltpu.VMEM((512, 16), jnp.float32),
            pltpu.VMEM_SHARED((ACC_ROWS, 16), jnp.float32),
        ] + [pltpu.SemaphoreType.DMA] * 5,
    )
    def kern(d_hbm, out_hbm, dbuf, obuf, zbuf, acc, i0, i1, i2, sa, sb):
        cid = lax.axis_index("c")
        sid = lax.axis_index("s")
        wid = sid * NC + cid
        base = wid * (nc * k)
        isem = (i0, i1, i2)
        ssem = (sa, sb)
        _fill_zeros(zbuf)
        _zero_acc(acc, zbuf, sid, sa)

        @pl.loop(0, k)
        def _(i):
            obuf[i, :] = jnp.ones((16,), jnp.float32)

        plsc.subcore_barrier()
        ih = {0: pltpu.async_copy(d_hbm.at[pl.ds(base, k)], dbuf.at[0], i0)}
        sh = {}
        for c in range(nc):
            b2, b3 = c % 2, c % 3
            if c >= 2:
                sh[c - 2].wait()
            if c + 1 < nc:
                nb3 = (c + 1) % 3
                ih[c + 1] = pltpu.async_copy(
                    d_hbm.at[pl.ds(base + (c + 1) * k, k)], dbuf.at[nb3],
                    isem[nb3])
            ih.pop(c).wait()
            sh[c] = pltpu.async_copy(obuf, acc.at[dbuf.at[b3]], ssem[b2],
                                     add=True)
        for c in range(max(0, nc - 2), nc):
            sh[c].wait()
        plsc.subcore_barrier()
        _acc_to_out(acc, zbuf, out_hbm, cid, sid, (i0, i1), (sa, sb))

    return kern


@functools.cache
def _sc_gather_segadd(nc, k):
    @functools.partial(
        pl.kernel,
        out_type=jax.ShapeDtypeStruct((NC, ACC_ROWS, 16), jnp.float32),
        mesh=_sc_mesh(),
        compiler_params=_SC_PARAMS,
        scratch_types=[
            pltpu.VMEM((3, k), jnp.int32),
            pltpu.VMEM((3, k), jnp.int32),
            pltpu.VMEM((2, k, 16), jnp.float32),
            pltpu.VMEM((512, 16), jnp.float32),
            pltpu.VMEM_SHARED((ACC_ROWS, 16), jnp.float32),
        ] + [pltpu.SemaphoreType.DMA] * 7,
    )
    def kern(tab_hbm, s_hbm, d_hbm, out_hbm, sbuf, dbuf, gbuf, zbuf, acc,
             i0, i1, i2, g0, g1, sa, sb):
        cid = lax.axis_index("c")
        sid = lax.axis_index("s")
        wid = sid * NC + cid
        base = wid * (nc * k)
        isem = (i0, i1, i2)
        gsem = (g0, g1)
        ssem = (sa, sb)
        _fill_zeros(zbuf)
        _zero_acc(acc, zbuf, sid, g0)
        plsc.subcore_barrier()

        def start_idx(c):
            b3 = c % 3
            off = base + c * k
            return (pltpu.async_copy(s_hbm.at[pl.ds(off, k)], sbuf.at[b3],
                                     isem[b3]),
                    pltpu.async_copy(d_hbm.at[pl.ds(off, k)], dbuf.at[b3],
                                     isem[b3]))

        ih = {0: start_idx(0)}
        sh = {}
        for c in range(nc):
            b2, b3 = c % 2, c % 3
            if c >= 2:
                sh[c - 2].wait()
            if c + 1 < nc:
                ih[c + 1] = start_idx(c + 1)
            for h in ih.pop(c):
                h.wait()
            pltpu.async_copy(tab_hbm.at[sbuf.at[b3]], gbuf.at[b2],
                             gsem[b2]).wait()
            sh[c] = pltpu.async_copy(gbuf.at[b2], acc.at[dbuf.at[b3]],
                                     ssem[b2], add=True)
        for c in range(max(0, nc - 2), nc):
            sh[c].wait()
        plsc.subcore_barrier()
        _acc_to_out(acc, zbuf, out_hbm, cid, sid, (i0, i1), (g0, g1))

    return kern


@functools.cache
def _sc_pair_gather(nc, k):
    epad = NW * nc * k
    @functools.partial(
        pl.kernel,
        out_type=(
            jax.ShapeDtypeStruct((epad, 32), jnp.float32),
            jax.ShapeDtypeStruct((epad, 32), jnp.float32),
            jax.ShapeDtypeStruct((epad, 16), jnp.float32),
        ),
        mesh=_sc_mesh(),
        compiler_params=_SC_PARAMS,
        scratch_types=[
            pltpu.VMEM((3, k), jnp.int32),
            pltpu.VMEM((3, k), jnp.int32),
            pltpu.VMEM((2, k, 32), jnp.float32),
            pltpu.VMEM((2, k, 32), jnp.float32),
            pltpu.VMEM((2, k, 16), jnp.float32),
        ] + [pltpu.SemaphoreType.DMA] * 15,
    )
    def kern(a_hbm, b_hbm, t_hbm, dst_hbm, src_hbm, pa_hbm, pb_hbm, ts_hbm,
             dbuf, sbuf, ga, gb, gt,
             i0, i1, i2, ga0, ga1, gb0, gb1, gt0, gt1,
             wa0, wa1, wb0, wb1, wt0, wt1):
        cid = lax.axis_index("c")
        sid = lax.axis_index("s")
        wid = sid * NC + cid
        base = wid * (nc * k)
        isem = (i0, i1, i2)
        gas, gbs, gts = (ga0, ga1), (gb0, gb1), (gt0, gt1)
        was, wbs, wts = (wa0, wa1), (wb0, wb1), (wt0, wt1)

        def start_idx(c):
            b3 = c % 3
            off = base + c * k
            return (pltpu.async_copy(dst_hbm.at[pl.ds(off, k)], dbuf.at[b3],
                                     isem[b3]),
                    pltpu.async_copy(src_hbm.at[pl.ds(off, k)], sbuf.at[b3],
                                     isem[b3]))

        ih = {0: start_idx(0)}
        wh = {}
        for c in range(nc):
            b2, b3 = c % 2, c % 3
            off = base + c * k
            if c >= 2:
                for h in wh.pop(c - 2):
                    h.wait()
            if c + 1 < nc:
                ih[c + 1] = start_idx(c + 1)
            for h in ih.pop(c):
                h.wait()
            ha = pltpu.async_copy(a_hbm.at[dbuf.at[b3]], ga.at[b2], gas[b2])
            hb = pltpu.async_copy(b_hbm.at[sbuf.at[b3]], gb.at[b2], gbs[b2])
            ht = pltpu.async_copy(t_hbm.at[sbuf.at[b3]], gt.at[b2], gts[b2])
            ha.wait()
            oa = pltpu.async_copy(ga.at[b2], pa_hbm.at[pl.ds(off, k)], was[b2])
            hb.wait()
            ob = pltpu.async_copy(gb.at[b2], pb_hbm.at[pl.ds(off, k)], wbs[b2])
            ht.wait()
            ot = pltpu.async_copy(gt.at[b2], ts_hbm.at[pl.ds(off, k)], wts[b2])
            wh[c] = (oa, ob, ot)
        for c in range(max(0, nc - 2), nc):
            for h in wh[c]:
                h.wait()

    return kern


@functools.cache
def _sc_segadd_vals(nc, k):
    @functools.partial(
        pl.kernel,
        out_type=jax.ShapeDtypeStruct((NC, ACC_ROWS, 16), jnp.float32),
        mesh=_sc_mesh(),
        compiler_params=_SC_PARAMS,
        scratch_types=[
            pltpu.VMEM((3, k), jnp.int32),
            pltpu.VMEM((3, k, 16), jnp.float32),
            pltpu.VMEM((384, 16), jnp.float32),
            pltpu.VMEM_SHARED((ACC_ROWS, 16), jnp.float32),
        ] + [pltpu.SemaphoreType.DMA] * 5,
    )
    def kern(v_hbm, d_hbm, out_hbm, dbuf, vbuf, zbuf, acc, i0, i1, i2, sa, sb):
        cid = lax.axis_index("c")
        sid = lax.axis_index("s")
        wid = sid * NC + cid
        base = wid * (nc * k)
        isem = (i0, i1, i2)
        ssem = (sa, sb)
        _fill_zeros(zbuf)
        _zero_acc(acc, zbuf, sid, sa)
        plsc.subcore_barrier()

        def start_in(c):
            b3 = c % 3
            off = base + c * k
            return (pltpu.async_copy(v_hbm.at[pl.ds(off, k)], vbuf.at[b3],
                                     isem[b3]),
                    pltpu.async_copy(d_hbm.at[pl.ds(off, k)], dbuf.at[b3],
                                     isem[b3]))

        ih = {0: start_in(0)}
        sh = {}
        for c in range(nc):
            b2, b3 = c % 2, c % 3
            if c >= 2:
                sh[c - 2].wait()
            if c + 1 < nc:
                ih[c + 1] = start_in(c + 1)
            for h in ih.pop(c):
                h.wait()
            sh[c] = pltpu.async_copy(vbuf.at[b3], acc.at[dbuf.at[b3]],
                                     ssem[b2], add=True)
        for c in range(max(0, nc - 2), nc):
            sh[c].wait()
        plsc.subcore_barrier()
        _acc_to_out(acc, zbuf, out_hbm, cid, sid, (i0, i1), (sa, sb))

    return kern


def _elu(v):
    return jnp.where(v > 0, v, jnp.exp(v) - 1.0)


def _dot(a, b):
    return jnp.dot(a, b, preferred_element_type=jnp.float32)


def _full(spec_shape):
    return pl.BlockSpec(spec_shape, lambda i: tuple(0 for _ in spec_shape))


def _rows(d):
    return pl.BlockSpec((_BLK, d), lambda i: (i, 0))


def _prows(d):
    return pl.BlockSpec((NC, _BLK, d), lambda i: (0, i, 0))


def _tc_prep(x, ego1_W, ego1_b, ego2_W, ego2_b, gcn1_W):
    def body(x_r, w1_r, b1_r, w2_r, b2_r, g1_r, hego_r, xw1_r):
        xb = x_r[...]
        t = _elu(_dot(xb, w1_r[...]) + b1_r[...])
        hego_r[...] = _dot(t, w2_r[...]) + b2_r[...]
        xw1_r[...] = _dot(xb, g1_r[...])

    return pl.pallas_call(
        body,
        grid=(N // _BLK,),
        in_specs=[_rows(COV), _full((COV, H)), _full((1, H)), _full((H, H)),
                  _full((1, H)), _full((COV, H))],
        out_specs=[_rows(H), _rows(H)],
        out_shape=[jax.ShapeDtypeStruct((N, H), jnp.float32),
                   jax.ShapeDtypeStruct((N, H), jnp.float32)],
    )(x, ego1_W, ego1_b, ego2_W, ego2_b, gcn1_W)


def _tc_dinv_y1(cnt, xw1):
    def body(cnt_r, xw1_r, dinv_r, y1a_r, y1b_r):
        deg = 1.0 + cnt_r[0, :, 0:1] + cnt_r[1, :, 0:1]
        dinv = lax.rsqrt(deg)
        y1 = xw1_r[...] * dinv
        dinv_r[...] = dinv
        y1a_r[...] = y1[:, :16]
        y1b_r[...] = y1[:, 16:]

    return pl.pallas_call(
        body,
        grid=(N // _BLK,),
        in_specs=[_prows(16), _rows(H)],
        out_specs=[_rows(1), _rows(16), _rows(16)],
        out_shape=[jax.ShapeDtypeStruct((N, 1), jnp.float32),
                   jax.ShapeDtypeStruct((N, 16), jnp.float32),
                   jax.ShapeDtypeStruct((N, 16), jnp.float32)],
    )(cnt, xw1)


def _tc_gcn_post(pa, pb, xw, dinv, b, ln_g, ln_b, next_W, emit_y):
    def body(pa_r, pb_r, xw_r, dinv_r, b_r, g_r, lb_r, w_r, *outs):
        g1 = jnp.concatenate(
            [pa_r[0] + pa_r[1], pb_r[0] + pb_r[1]], axis=1)
        dinv = dinv_r[...]
        z = dinv * g1 + (dinv * dinv) * xw_r[...] + b_r[...]
        mu = jnp.mean(z, axis=-1, keepdims=True)
        var = jnp.mean((z - mu) ** 2, axis=-1, keepdims=True)
        hn = (z - mu) * lax.rsqrt(var + 1e-5) * g_r[...] + lb_r[...]
        h = _elu(hn)
        xw2 = _dot(h, w_r[...])
        if emit_y:
            h_r, xw2_r, ya_r, yb_r = outs
            y = xw2 * dinv
            ya_r[...] = y[:, :16]
            yb_r[...] = y[:, 16:]
        else:
            h_r, xw2_r = outs
        h_r[...] = h
        xw2_r[...] = xw2

    out_specs = [_rows(H), _rows(H)]
    out_shape = [jax.ShapeDtypeStruct((N, H), jnp.float32),
                 jax.ShapeDtypeStruct((N, H), jnp.float32)]
    if emit_y:
        out_specs += [_rows(16), _rows(16)]
        out_shape += [jax.ShapeDtypeStruct((N, 16), jnp.float32),
                      jax.ShapeDtypeStruct((N, 16), jnp.float32)]
    return pl.pallas_call(
        body,
        grid=(N // _BLK,),
        in_specs=[_prows(16), _prows(16), _rows(H), _rows(1), _full((1, H)),
                  _full((1, H)), _full((1, H)), _full((H, H))],
        out_specs=out_specs,
        out_shape=out_shape,
    )(pa, pb, xw, dinv, b, ln_g, ln_b, next_W)


def _tc_ab(h_ego, h_gnn, wn_Ws, wn_bs):
    def body(he_r, hg_r, *rest):
        ws = rest[:3]
        bs = rest[3:6]
        outs = rest[6:]
        he = he_r[...]
        hg = hg_r[...]
        for t in range(3):
            w = ws[t][...]
            outs[2 * t][...] = (_dot(he, w[0:H]) + _dot(hg, w[H:2 * H])
                                + bs[t][...])
            outs[2 * t + 1][...] = (_dot(he, w[2 * H:3 * H])
                                    + _dot(hg, w[3 * H:]))

    return pl.pallas_call(
        body,
        grid=(N // _BLK,),
        in_specs=[_rows(H), _rows(H)] + [_full((4 * H, H))] * 3
                 + [_full((1, H))] * 3,
        out_specs=[_rows(H)] * 6,
        out_shape=[jax.ShapeDtypeStruct((N, H), jnp.float32)] * 6,
    )(h_ego, h_gnn, *wn_Ws, *wn_bs)


def _tc_edge_mlp(pa, pb, ts, w2, b2, blk):
    epad = pa.shape[0]

    def body(pa_r, pb_r, ts_r, w2_r, b2_r, u_r):
        hid = _elu(pa_r[...] + pb_r[...])
        logit = _dot(hid, w2_r[...]) + b2_r[...]
        w = jax.nn.sigmoid(logit)
        tcols = ts_r[...]
        u = jnp.concatenate([tcols[:, :T] * w, w,
                             jnp.zeros((blk, 16 - T - 1), jnp.float32)],
                            axis=1)
        u_r[...] = u

    return pl.pallas_call(
        body,
        grid=(epad // blk,),
        in_specs=[pl.BlockSpec((blk, H), lambda i: (i, 0)),
                  pl.BlockSpec((blk, H), lambda i: (i, 0)),
                  pl.BlockSpec((blk, 16), lambda i: (i, 0)),
                  _full((H, 1)), _full((1, 1))],
        out_specs=[pl.BlockSpec((blk, 16), lambda i: (i, 0))],
        out_shape=[jax.ShapeDtypeStruct((epad, 16), jnp.float32)],
    )(pa, pb, ts, w2, b2)


def _tc_heads(h_ego, h_gnn, treat, qs, p):
    def body(he_r, hg_r, tr_r, q0_r, q1_r, q2_r,
             o1w_r, o1b_r, o2w_r, o2b_r, o3w_r, o3b_r,
             l1w_r, l1b_r, l2w_r, l2b_r,
             m1w_r, m1b_r, m2w_r, m2b_r,
             v1w_r, v1b_r, v2w_r, v2b_r,
             yf_r, yl_r, ex_r, mu_r, lv_r):
        he = he_r[...]
        hg = hg_r[...]
        tr = tr_r[...]
        exps = []
        for q_r in (q0_r, q1_r, q2_r):
            q = q_r[0] + q_r[1]
            exps.append(q[:, :T] / jnp.maximum(q[:, T:T + 1], 1e-8))
        exposure = jnp.concatenate(exps, axis=1)
        ex_r[...] = exposure
        z = jnp.concatenate([he, hg, tr, exposure], axis=1)
        y = _elu(_dot(z, o1w_r[...]) + o1b_r[...])
        y = _elu(_dot(y, o2w_r[...]) + o2b_r[...])
        y = _dot(y, o3w_r[...]) + o3b_r[...]
        y = y - jnp.max(y, axis=-1, keepdims=True)
        ey = jnp.exp(y)
        yf_r[...] = ey / jnp.sum(ey, axis=-1, keepdims=True)
        zl = jnp.concatenate([he, tr], axis=1)
        y2 = _elu(_dot(zl, l1w_r[...]) + l1b_r[...])
        y2 = _dot(y2, l2w_r[...]) + l2b_r[...]
        y2 = y2 - jnp.max(y2, axis=-1, keepdims=True)
        ey2 = jnp.exp(y2)
        yl_r[...] = ey2 / jnp.sum(ey2, axis=-1, keepdims=True)
        hh = jnp.concatenate([he, hg], axis=1)
        mu_r[...] = _dot(_elu(_dot(hh, m1w_r[...]) + m1b_r[...]),
                         m2w_r[...]) + m2b_r[...]
        lv_r[...] = _dot(_elu(_dot(hh, v1w_r[...]) + v1b_r[...]),
                         v2w_r[...]) + v2b_r[...]

    def _b(name):
        return p[name].reshape(1, -1)

    return pl.pallas_call(
        body,
        grid=(N // _BLK,),
        in_specs=[_rows(H), _rows(H), _rows(T)] + [_prows(16)] * 3
                 + [_full((2 * H + 4 * T, H)), _full((1, H)),
                    _full((H, H)), _full((1, H)),
                    _full((H, OUT)), _full((1, OUT)),
                    _full((H + T, H)), _full((1, H)),
                    _full((H, OUT)), _full((1, OUT)),
                    _full((2 * H, H)), _full((1, H)),
                    _full((H, T)), _full((1, T)),
                    _full((2 * H, H)), _full((1, H)),
                    _full((H, T)), _full((1, T))],
        out_specs=[_rows(OUT), _rows(OUT), _rows(3 * T), _rows(T), _rows(T)],
        out_shape=[jax.ShapeDtypeStruct((N, OUT), jnp.float32),
                   jax.ShapeDtypeStruct((N, OUT), jnp.float32),
                   jax.ShapeDtypeStruct((N, 3 * T), jnp.float32),
                   jax.ShapeDtypeStruct((N, T), jnp.float32),
                   jax.ShapeDtypeStruct((N, T), jnp.float32)],
    )(h_ego, h_gnn, treat, *qs,
      p['out1_W'], _b('out1_b'), p['out2_W'], _b('out2_b'),
      p['out3_W'], _b('out3_b'),
      p['loc1_W'], _b('loc1_b'), p['loc2_W'], _b('loc2_b'),
      p['gmu1_W'], _b('gmu1_b'), p['gmu2_W'], _b('gmu2_b'),
      p['glv1_W'], _b('glv1_b'), p['glv2_W'], _b('glv2_b'))


def _pad_idx(idx, epad, fill):
    return jnp.concatenate(
        [idx, jnp.full((epad - idx.shape[0],), fill, jnp.int32)])


def kernel(x, params, edge_index_0, edge_index_1, edge_index_2):
    p = params
    eis = [edge_index_0, edge_index_1, edge_index_2]

    nc3, k3, epad3 = _pad_plan(3 * E, 512)
    nc1, k1, epad1 = _pad_plan(E, 448)

    s_all = _pad_idx(jnp.concatenate([ei[0] for ei in eis]), epad3, 0)
    d_all = _pad_idx(jnp.concatenate([ei[1] for ei in eis]), epad3, N)

    treat = x[:, :T]
    treat16 = jnp.concatenate(
        [treat, jnp.zeros((N, 16 - T), jnp.float32)], axis=1)

    cnt = _sc_count(nc3, k3)(d_all)
    h_ego, xw1 = _tc_prep(
        x, p['ego1_W'], p['ego1_b'].reshape(1, H), p['ego2_W'],
        p['ego2_b'].reshape(1, H), p['gcn1_W'])

    dinv, y1a, y1b = _tc_dinv_y1(cnt, xw1)

    seg = _sc_gather_segadd(nc3, k3)
    p1a = seg(y1a, s_all, d_all)
    p1b = seg(y1b, s_all, d_all)
    h1, xw2, y2a, y2b = _tc_gcn_post(
        p1a, p1b, xw1, dinv, p['gcn1_b'].reshape(1, H), p['ln1_g'].reshape(1, H),
        p['ln1_b'].reshape(1, H), p['gcn2_W'], emit_y=True)

    p2a = seg(y2a, s_all, d_all)
    p2b = seg(y2b, s_all, d_all)
    h_gnn, _, = _tc_gcn_post(
        p2a, p2b, xw2, dinv, p['gcn2_b'].reshape(1, H), p['ln2_g'].reshape(1, H),
        p['ln2_b'].reshape(1, H), p['gcn2_W'], emit_y=False)[:2]

    ab = _tc_ab(h_gnn=h_gnn, h_ego=h_ego,
                wn_Ws=[p['wn%d_1W' % t] for t in range(3)],
                wn_bs=[p['wn%d_1b' % t].reshape(1, H) for t in range(3)])

    pair = _sc_pair_gather(nc1, k1)
    segv = _sc_segadd_vals(nc1, k1)
    weights = []
    qs = []
    for t in range(3):
        dst = _pad_idx(eis[t][1], epad1, N)
        src = _pad_idx(eis[t][0], epad1, 0)
        pa, pb, ts = pair(ab[2 * t], ab[2 * t + 1], treat16, dst, src)
        u_t, = _tc_edge_mlp(
            pa, pb, ts, p['wn%d_2W' % t],
            p['wn%d_2b' % t].reshape(1, 1), blk=2048)
        weights.append(u_t[:E, T])
        qs.append(segv(u_t, dst))

    y_full, y_local, exposure, gps_mu, gps_logvar = _tc_heads(
        h_ego, h_gnn, treat, qs, p)

    return (y_full, y_local, exposure, tuple(weights), gps_mu, gps_logvar)


# speedup vs baseline: 1.3338x; 1.3338x over previous
import functools
import math

import jax
import jax.numpy as jnp
from jax import lax
from jax.experimental import pallas as pl
from jax.experimental.pallas import tpu as pltpu
from jax.experimental.pallas import tpu_sc as plsc

N = 100000
COV = 128
T = 4
OUT = 5
H = 32
E = 200000

NC = 2
NS = 16
NW = NC * NS

ACC_ROWS = 100096
_SUB_ROWS = ACC_ROWS // NS
_ZROWS = 256

_BLK = 2000


def _pad_plan(num, k):
    nc = max(1, math.ceil(num / (NW * k)))
    return nc, k, NW * nc * k


def _sc_mesh():
    return plsc.VectorSubcoreMesh(core_axis_name="c", subcore_axis_name="s")


_SC_PARAMS = pltpu.CompilerParams(use_tc_tiling_on_sc=False)
_SC_PARAMS_NL = pltpu.CompilerParams(use_tc_tiling_on_sc=False,
                                     needs_layout_passes=False)


def _fill_zeros(zbuf):
    @pl.loop(0, zbuf.shape[0])
    def _(i):
        zbuf[i, :] = jnp.zeros((16,), jnp.float32)


def _zero_acc(acc, zbuf, sid, sem):
    base = sid * _SUB_ROWS
    zr = zbuf.shape[0] // 2
    hs = []
    off = 0
    while off < _SUB_ROWS:
        rows = min(zr, _SUB_ROWS - off)
        hs.append(pltpu.async_copy(
            zbuf.at[pl.ds(0, rows)], acc.at[pl.ds(base + off, rows)], sem))
        if len(hs) >= 5:
            hs[-5].wait()
        off += rows
    for h in hs[-4:]:
        h.wait()


def _acc_to_out(acc, zbuf, out_hbm, cid, sid, sem_r, sem_w):
    base = sid * _SUB_ROWS
    zr = zbuf.shape[0] // 2
    wh = {}
    j = 0
    off = 0
    while off < _SUB_ROWS:
        rows = min(zr, _SUB_ROWS - off)
        b = j & 1
        if j >= 2:
            wh[j - 2].wait()
        pltpu.async_copy(acc.at[pl.ds(base + off, rows)],
                         zbuf.at[pl.ds(b * zr, rows)], sem_r[b]).wait()
        wh[j] = pltpu.async_copy(zbuf.at[pl.ds(b * zr, rows)],
                                 out_hbm.at[cid, pl.ds(base + off, rows)],
                                 sem_w[b])
        j += 1
        off += rows
    for jj in range(max(0, j - 2), j):
        wh[jj].wait()


@functools.cache
def _sc_count(nc, k):
    @functools.partial(
        pl.kernel,
        out_type=jax.ShapeDtypeStruct((NC, ACC_ROWS, 16), jnp.float32),
        mesh=_sc_mesh(),
        compiler_params=_SC_PARAMS,
        scratch_types=[
            pltpu.VMEM((3, k), jnp.int32),
            pltpu.VMEM((k, 16), jnp.float32),
            pltpu.VMEM((512, 16), jnp.float32),
            pltpu.VMEM_SHARED((ACC_ROWS, 16), jnp.float32),
        ] + [pltpu.SemaphoreType.DMA] * 5,
    )
    def kern(d_hbm, out_hbm, dbuf, obuf, zbuf, acc, i0, i1, i2, sa, sb):
        cid = lax.axis_index("c")
        sid = lax.axis_index("s")
        wid = sid * NC + cid
        base = wid * (nc * k)
        isem = (i0, i1, i2)
        ssem = (sa, sb)
        _fill_zeros(zbuf)
        _zero_acc(acc, zbuf, sid, sa)

        @pl.loop(0, k)
        def _(i):
            obuf[i, :] = jnp.ones((16,), jnp.float32)

        plsc.subcore_barrier()
        ih = {0: pltpu.async_copy(d_hbm.at[pl.ds(base, k)], dbuf.at[0], i0)}
        sh = {}
        for c in range(nc):
            b2, b3 = c % 2, c % 3
            if c >= 2:
                sh[c - 2].wait()
            if c + 1 < nc:
                nb3 = (c + 1) % 3
                ih[c + 1] = pltpu.async_copy(
                    d_hbm.at[pl.ds(base + (c + 1) * k, k)], dbuf.at[nb3],
                    isem[nb3])
            ih.pop(c).wait()
            sh[c] = pltpu.async_copy(obuf, acc.at[dbuf.at[b3]], ssem[b2],
                                     add=True)
        for c in range(max(0, nc - 2), nc):
            sh[c].wait()
        plsc.subcore_barrier()
        _acc_to_out(acc, zbuf, out_hbm, cid, sid, (i0, i1), (sa, sb))

    return kern


@functools.cache
def _sc_gather_segadd(nc, k):
    @functools.partial(
        pl.kernel,
        out_type=jax.ShapeDtypeStruct((NC, ACC_ROWS, 16), jnp.float32),
        mesh=_sc_mesh(),
        compiler_params=_SC_PARAMS,
        scratch_types=[
            pltpu.VMEM((3, k), jnp.int32),
            pltpu.VMEM((3, k), jnp.int32),
            pltpu.VMEM((2, k, 16), jnp.float32),
            pltpu.VMEM((512, 16), jnp.float32),
            pltpu.VMEM_SHARED((ACC_ROWS, 16), jnp.float32),
        ] + [pltpu.SemaphoreType.DMA] * 7,
    )
    def kern(tab_hbm, s_hbm, d_hbm, out_hbm, sbuf, dbuf, gbuf, zbuf, acc,
             i0, i1, i2, g0, g1, sa, sb):
        cid = lax.axis_index("c")
        sid = lax.axis_index("s")
        wid = sid * NC + cid
        base = wid * (nc * k)
        isem = (i0, i1, i2)
        gsem = (g0, g1)
        ssem = (sa, sb)
        _fill_zeros(zbuf)
        _zero_acc(acc, zbuf, sid, g0)
        plsc.subcore_barrier()

        def start_idx(c):
            b3 = c % 3
            off = base + c * k
            return (pltpu.async_copy(s_hbm.at[pl.ds(off, k)], sbuf.at[b3],
                                     isem[b3]),
                    pltpu.async_copy(d_hbm.at[pl.ds(off, k)], dbuf.at[b3],
                                     isem[b3]))

        ih = {0: start_idx(0)}
        sh = {}
        for c in range(nc):
            b2, b3 = c % 2, c % 3
            if c >= 2:
                sh[c - 2].wait()
            if c + 1 < nc:
                ih[c + 1] = start_idx(c + 1)
            for h in ih.pop(c):
                h.wait()
            pltpu.async_copy(tab_hbm.at[sbuf.at[b3]], gbuf.at[b2],
                             gsem[b2]).wait()
            sh[c] = pltpu.async_copy(gbuf.at[b2], acc.at[dbuf.at[b3]],
                                     ssem[b2], add=True)
        for c in range(max(0, nc - 2), nc):
            sh[c].wait()
        plsc.subcore_barrier()
        _acc_to_out(acc, zbuf, out_hbm, cid, sid, (i0, i1), (g0, g1))

    return kern


@functools.cache
def _sc_edge_mlp(nc, k):
    epad = NW * nc * k
    @functools.partial(
        pl.kernel,
        out_type=(
            jax.ShapeDtypeStruct((epad, 16), jnp.float32),
            jax.ShapeDtypeStruct((epad,), jnp.float32),
        ),
        mesh=_sc_mesh(),
        compiler_params=_SC_PARAMS_NL,
        scratch_types=[
            pltpu.VMEM((3, k), jnp.int32),
            pltpu.VMEM((3, k), jnp.int32),
            pltpu.VMEM((2, k, 32), jnp.float32),
            pltpu.VMEM((2, k, 32), jnp.float32),
            pltpu.VMEM((2, k, 16), jnp.float32),
            pltpu.VMEM((2, k, 16), jnp.float32),
            pltpu.VMEM((2, k), jnp.float32),
            pltpu.VMEM((32, 16), jnp.float32),
            pltpu.VMEM((16,), jnp.float32),
        ] + [pltpu.SemaphoreType.DMA] * 13,
    )
    def kern(a_hbm, b_hbm, t_hbm, dst_hbm, src_hbm, w2_hbm, b2_hbm,
             u_hbm, w_hbm,
             dbuf, sbuf, ga, gb, gt, ubuf, wbuf, w2buf, b2buf,
             i0, i1, i2, ga0, ga1, gb0, gb1, gt0, gt1, su0, su1, sw0, sw1):
        cid = lax.axis_index("c")
        sid = lax.axis_index("s")
        wid = sid * NC + cid
        base = wid * (nc * k)
        isem = (i0, i1, i2)
        gas, gbs, gts = (ga0, ga1), (gb0, gb1), (gt0, gt1)
        sus, sws = (su0, su1), (sw0, sw1)
        pltpu.sync_copy(w2_hbm, w2buf)
        pltpu.sync_copy(b2_hbm, b2buf)
        b2v = b2buf[...]
        lanes = lax.iota(jnp.int32, 16)

        def start_idx(c):
            b3 = c % 3
            off = base + c * k
            return (pltpu.async_copy(dst_hbm.at[pl.ds(off, k)], dbuf.at[b3],
                                     isem[b3]),
                    pltpu.async_copy(src_hbm.at[pl.ds(off, k)], sbuf.at[b3],
                                     isem[b3]))

        gh = {}

        def start_gather(c):
            b2, b3 = c % 2, c % 3
            gh[c] = (
                pltpu.async_copy(a_hbm.at[dbuf.at[b3]], ga.at[b2], gas[b2]),
                pltpu.async_copy(b_hbm.at[sbuf.at[b3]], gb.at[b2], gbs[b2]),
                pltpu.async_copy(t_hbm.at[sbuf.at[b3]], gt.at[b2], gts[b2]),
            )

        def compute(cc):
            b2 = cc % 2
            gar = ga.at[b2]
            gbr = gb.at[b2]

            @pl.loop(0, k, step=16)
            def _(row0):
                ridx = row0 + lanes

                def feat(j, acc):
                    cidx = jnp.full((16,), j, jnp.int32)
                    s = (plsc.load_gather(gar, [ridx, cidx])
                         + plsc.load_gather(gbr, [ridx, cidx]))
                    e = jnp.where(s > 0, s, jnp.exp(s) - 1.0)
                    return acc + e * w2buf[j, :]

                acc = lax.fori_loop(0, 32, feat, b2v, unroll=4)
                wv = 1.0 / (1.0 + jnp.exp(-acc))
                wbuf[b2, pl.ds(row0, 16)] = wv
                for ei in range(16):
                    trow = gt[b2, row0 + ei, :]
                    ubuf[b2, row0 + ei, :] = trow * wv[ei]

        ih = {0: start_idx(0)}
        wh = {}
        for c in range(nc + 1):
            if c < nc:
                b2 = c % 2
                if c >= 2:
                    for h in wh.pop(c - 2):
                        h.wait()
                for h in ih.pop(c):
                    h.wait()
                if c + 1 < nc:
                    ih[c + 1] = start_idx(c + 1)
                start_gather(c)
            if c >= 1:
                cc = c - 1
                b2 = cc % 2
                off = base + cc * k
                for h in gh.pop(cc):
                    h.wait()
                compute(cc)
                wh[cc] = (
                    pltpu.async_copy(ubuf.at[b2], u_hbm.at[pl.ds(off, k)],
                                     sus[b2]),
                    pltpu.async_copy(wbuf.at[b2], w_hbm.at[pl.ds(off, k)],
                                     sws[b2]),
                )
        for c in range(max(0, nc - 2), nc):
            for h in wh[c]:
                h.wait()

    return kern


@functools.cache
def _sc_segadd_vals(nc, k):
    @functools.partial(
        pl.kernel,
        out_type=jax.ShapeDtypeStruct((NC, ACC_ROWS, 16), jnp.float32),
        mesh=_sc_mesh(),
        compiler_params=_SC_PARAMS,
        scratch_types=[
            pltpu.VMEM((3, k), jnp.int32),
            pltpu.VMEM((3, k, 16), jnp.float32),
            pltpu.VMEM((384, 16), jnp.float32),
            pltpu.VMEM_SHARED((ACC_ROWS, 16), jnp.float32),
        ] + [pltpu.SemaphoreType.DMA] * 5,
    )
    def kern(v_hbm, d_hbm, out_hbm, dbuf, vbuf, zbuf, acc, i0, i1, i2, sa, sb):
        cid = lax.axis_index("c")
        sid = lax.axis_index("s")
        wid = sid * NC + cid
        base = wid * (nc * k)
        isem = (i0, i1, i2)
        ssem = (sa, sb)
        _fill_zeros(zbuf)
        _zero_acc(acc, zbuf, sid, sa)
        plsc.subcore_barrier()

        def start_in(c):
            b3 = c % 3
            off = base + c * k
            return (pltpu.async_copy(v_hbm.at[pl.ds(off, k)], vbuf.at[b3],
                                     isem[b3]),
                    pltpu.async_copy(d_hbm.at[pl.ds(off, k)], dbuf.at[b3],
                                     isem[b3]))

        ih = {0: start_in(0)}
        sh = {}
        for c in range(nc):
            b2, b3 = c % 2, c % 3
            if c >= 2:
                sh[c - 2].wait()
            if c + 1 < nc:
                ih[c + 1] = start_in(c + 1)
            for h in ih.pop(c):
                h.wait()
            sh[c] = pltpu.async_copy(vbuf.at[b3], acc.at[dbuf.at[b3]],
                                     ssem[b2], add=True)
        for c in range(max(0, nc - 2), nc):
            sh[c].wait()
        plsc.subcore_barrier()
        _acc_to_out(acc, zbuf, out_hbm, cid, sid, (i0, i1), (sa, sb))

    return kern


def _elu(v):
    return jnp.where(v > 0, v, jnp.exp(v) - 1.0)


def _dot(a, b):
    return jnp.dot(a, b, preferred_element_type=jnp.float32)


def _full(spec_shape):
    return pl.BlockSpec(spec_shape, lambda i: tuple(0 for _ in spec_shape))


def _rows(d):
    return pl.BlockSpec((_BLK, d), lambda i: (i, 0))


def _prows(d):
    return pl.BlockSpec((NC, _BLK, d), lambda i: (0, i, 0))


def _tc_prep(x, ego1_W, ego1_b, ego2_W, ego2_b, gcn1_W):
    def body(x_r, w1_r, b1_r, w2_r, b2_r, g1_r, hego_r, xw1_r):
        xb = x_r[...]
        t = _elu(_dot(xb, w1_r[...]) + b1_r[...])
        hego_r[...] = _dot(t, w2_r[...]) + b2_r[...]
        xw1_r[...] = _dot(xb, g1_r[...])

    return pl.pallas_call(
        body,
        grid=(N // _BLK,),
        in_specs=[_rows(COV), _full((COV, H)), _full((1, H)), _full((H, H)),
                  _full((1, H)), _full((COV, H))],
        out_specs=[_rows(H), _rows(H)],
        out_shape=[jax.ShapeDtypeStruct((N, H), jnp.float32),
                   jax.ShapeDtypeStruct((N, H), jnp.float32)],
    )(x, ego1_W, ego1_b, ego2_W, ego2_b, gcn1_W)


def _tc_dinv_y1(cnt, xw1):
    def body(cnt_r, xw1_r, dinv_r, y1a_r, y1b_r):
        deg = 1.0 + cnt_r[0, :, 0:1] + cnt_r[1, :, 0:1]
        dinv = lax.rsqrt(deg)
        y1 = xw1_r[...] * dinv
        dinv_r[...] = dinv
        y1a_r[...] = y1[:, :16]
        y1b_r[...] = y1[:, 16:]

    return pl.pallas_call(
        body,
        grid=(N // _BLK,),
        in_specs=[_prows(16), _rows(H)],
        out_specs=[_rows(1), _rows(16), _rows(16)],
        out_shape=[jax.ShapeDtypeStruct((N, 1), jnp.float32),
                   jax.ShapeDtypeStruct((N, 16), jnp.float32),
                   jax.ShapeDtypeStruct((N, 16), jnp.float32)],
    )(cnt, xw1)


def _tc_gcn_post(pa, pb, xw, dinv, b, ln_g, ln_b, next_W, emit_y):
    def body(pa_r, pb_r, xw_r, dinv_r, b_r, g_r, lb_r, w_r, *outs):
        g1 = jnp.concatenate(
            [pa_r[0] + pa_r[1], pb_r[0] + pb_r[1]], axis=1)
        dinv = dinv_r[...]
        z = dinv * g1 + (dinv * dinv) * xw_r[...] + b_r[...]
        mu = jnp.mean(z, axis=-1, keepdims=True)
        var = jnp.mean((z - mu) ** 2, axis=-1, keepdims=True)
        hn = (z - mu) * lax.rsqrt(var + 1e-5) * g_r[...] + lb_r[...]
        h = _elu(hn)
        xw2 = _dot(h, w_r[...])
        if emit_y:
            h_r, xw2_r, ya_r, yb_r = outs
            y = xw2 * dinv
            ya_r[...] = y[:, :16]
            yb_r[...] = y[:, 16:]
        else:
            h_r, xw2_r = outs
        h_r[...] = h
        xw2_r[...] = xw2

    out_specs = [_rows(H), _rows(H)]
    out_shape = [jax.ShapeDtypeStruct((N, H), jnp.float32),
                 jax.ShapeDtypeStruct((N, H), jnp.float32)]
    if emit_y:
        out_specs += [_rows(16), _rows(16)]
        out_shape += [jax.ShapeDtypeStruct((N, 16), jnp.float32),
                      jax.ShapeDtypeStruct((N, 16), jnp.float32)]
    return pl.pallas_call(
        body,
        grid=(N // _BLK,),
        in_specs=[_prows(16), _prows(16), _rows(H), _rows(1), _full((1, H)),
                  _full((1, H)), _full((1, H)), _full((H, H))],
        out_specs=out_specs,
        out_shape=out_shape,
    )(pa, pb, xw, dinv, b, ln_g, ln_b, next_W)


def _tc_ab(h_ego, h_gnn, wn_Ws, wn_bs):
    def body(he_r, hg_r, *rest):
        ws = rest[:3]
        bs = rest[3:6]
        outs = rest[6:]
        he = he_r[...]
        hg = hg_r[...]
        for t in range(3):
            w = ws[t][...]
            outs[2 * t][...] = (_dot(he, w[0:H]) + _dot(hg, w[H:2 * H])
                                + bs[t][...])
            outs[2 * t + 1][...] = (_dot(he, w[2 * H:3 * H])
                                    + _dot(hg, w[3 * H:]))

    return pl.pallas_call(
        body,
        grid=(N // _BLK,),
        in_specs=[_rows(H), _rows(H)] + [_full((4 * H, H))] * 3
                 + [_full((1, H))] * 3,
        out_specs=[_rows(H)] * 6,
        out_shape=[jax.ShapeDtypeStruct((N, H), jnp.float32)] * 6,
    )(h_ego, h_gnn, *wn_Ws, *wn_bs)


def _tc_edge_mlp(pa, pb, ts, w2, b2, blk):
    epad = pa.shape[0]

    def body(pa_r, pb_r, ts_r, w2_r, b2_r, u_r):
        hid = _elu(pa_r[...] + pb_r[...])
        logit = _dot(hid, w2_r[...]) + b2_r[...]
        w = jax.nn.sigmoid(logit)
        tcols = ts_r[...]
        u = jnp.concatenate([tcols[:, :T] * w, w,
                             jnp.zeros((blk, 16 - T - 1), jnp.float32)],
                            axis=1)
        u_r[...] = u

    return pl.pallas_call(
        body,
        grid=(epad // blk,),
        in_specs=[pl.BlockSpec((blk, H), lambda i: (i, 0)),
                  pl.BlockSpec((blk, H), lambda i: (i, 0)),
                  pl.BlockSpec((blk, 16), lambda i: (i, 0)),
                  _full((H, 1)), _full((1, 1))],
        out_specs=[pl.BlockSpec((blk, 16), lambda i: (i, 0))],
        out_shape=[jax.ShapeDtypeStruct((epad, 16), jnp.float32)],
    )(pa, pb, ts, w2, b2)


def _tc_heads(h_ego, h_gnn, treat, qs, p):
    def body(he_r, hg_r, tr_r, q0_r, q1_r, q2_r,
             o1w_r, o1b_r, o2w_r, o2b_r, o3w_r, o3b_r,
             l1w_r, l1b_r, l2w_r, l2b_r,
             m1w_r, m1b_r, m2w_r, m2b_r,
             v1w_r, v1b_r, v2w_r, v2b_r,
             yf_r, yl_r, ex_r, mu_r, lv_r):
        he = he_r[...]
        hg = hg_r[...]
        tr = tr_r[...]
        exps = []
        for q_r in (q0_r, q1_r, q2_r):
            q = q_r[0] + q_r[1]
            exps.append(q[:, :T] / jnp.maximum(q[:, T:T + 1], 1e-8))
        exposure = jnp.concatenate(exps, axis=1)
        ex_r[...] = exposure
        z = jnp.concatenate([he, hg, tr, exposure], axis=1)
        y = _elu(_dot(z, o1w_r[...]) + o1b_r[...])
        y = _elu(_dot(y, o2w_r[...]) + o2b_r[...])
        y = _dot(y, o3w_r[...]) + o3b_r[...]
        y = y - jnp.max(y, axis=-1, keepdims=True)
        ey = jnp.exp(y)
        yf_r[...] = ey / jnp.sum(ey, axis=-1, keepdims=True)
        zl = jnp.concatenate([he, tr], axis=1)
        y2 = _elu(_dot(zl, l1w_r[...]) + l1b_r[...])
        y2 = _dot(y2, l2w_r[...]) + l2b_r[...]
        y2 = y2 - jnp.max(y2, axis=-1, keepdims=True)
        ey2 = jnp.exp(y2)
        yl_r[...] = ey2 / jnp.sum(ey2, axis=-1, keepdims=True)
        hh = jnp.concatenate([he, hg], axis=1)
        mu_r[...] = _dot(_elu(_dot(hh, m1w_r[...]) + m1b_r[...]),
                         m2w_r[...]) + m2b_r[...]
        lv_r[...] = _dot(_elu(_dot(hh, v1w_r[...]) + v1b_r[...]),
                         v2w_r[...]) + v2b_r[...]

    def _b(name):
        return p[name].reshape(1, -1)

    return pl.pallas_call(
        body,
        grid=(N // _BLK,),
        in_specs=[_rows(H), _rows(H), _rows(T)] + [_prows(16)] * 3
                 + [_full((2 * H + 4 * T, H)), _full((1, H)),
                    _full((H, H)), _full((1, H)),
                    _full((H, OUT)), _full((1, OUT)),
                    _full((H + T, H)), _full((1, H)),
                    _full((H, OUT)), _full((1, OUT)),
                    _full((2 * H, H)), _full((1, H)),
                    _full((H, T)), _full((1, T)),
                    _full((2 * H, H)), _full((1, H)),
                    _full((H, T)), _full((1, T))],
        out_specs=[_rows(OUT), _rows(OUT), _rows(3 * T), _rows(T), _rows(T)],
        out_shape=[jax.ShapeDtypeStruct((N, OUT), jnp.float32),
                   jax.ShapeDtypeStruct((N, OUT), jnp.float32),
                   jax.ShapeDtypeStruct((N, 3 * T), jnp.float32),
                   jax.ShapeDtypeStruct((N, T), jnp.float32),
                   jax.ShapeDtypeStruct((N, T), jnp.float32)],
    )(h_ego, h_gnn, treat, *qs,
      p['out1_W'], _b('out1_b'), p['out2_W'], _b('out2_b'),
      p['out3_W'], _b('out3_b'),
      p['loc1_W'], _b('loc1_b'), p['loc2_W'], _b('loc2_b'),
      p['gmu1_W'], _b('gmu1_b'), p['gmu2_W'], _b('gmu2_b'),
      p['glv1_W'], _b('glv1_b'), p['glv2_W'], _b('glv2_b'))


def _pad_idx(idx, epad, fill):
    return jnp.concatenate(
        [idx, jnp.full((epad - idx.shape[0],), fill, jnp.int32)])


def kernel(x, params, edge_index_0, edge_index_1, edge_index_2):
    p = params
    eis = [edge_index_0, edge_index_1, edge_index_2]

    nc3, k3, epad3 = _pad_plan(3 * E, 512)
    nc1, k1, epad1 = _pad_plan(E, 448)

    s_all = _pad_idx(jnp.concatenate([ei[0] for ei in eis]), epad3, 0)
    d_all = _pad_idx(jnp.concatenate([ei[1] for ei in eis]), epad3, N)

    treat = x[:, :T]
    treat16 = jnp.concatenate(
        [treat, jnp.ones((N, 1), jnp.float32),
         jnp.zeros((N, 16 - T - 1), jnp.float32)], axis=1)

    cnt = _sc_count(nc3, k3)(d_all)
    h_ego, xw1 = _tc_prep(
        x, p['ego1_W'], p['ego1_b'].reshape(1, H), p['ego2_W'],
        p['ego2_b'].reshape(1, H), p['gcn1_W'])

    dinv, y1a, y1b = _tc_dinv_y1(cnt, xw1)

    seg = _sc_gather_segadd(nc3, k3)
    p1a = seg(y1a, s_all, d_all)
    p1b = seg(y1b, s_all, d_all)
    h1, xw2, y2a, y2b = _tc_gcn_post(
        p1a, p1b, xw1, dinv, p['gcn1_b'].reshape(1, H), p['ln1_g'].reshape(1, H),
        p['ln1_b'].reshape(1, H), p['gcn2_W'], emit_y=True)

    p2a = seg(y2a, s_all, d_all)
    p2b = seg(y2b, s_all, d_all)
    h_gnn, _, = _tc_gcn_post(
        p2a, p2b, xw2, dinv, p['gcn2_b'].reshape(1, H), p['ln2_g'].reshape(1, H),
        p['ln2_b'].reshape(1, H), p['gcn2_W'], emit_y=False)[:2]

    ab = _tc_ab(h_gnn=h_gnn, h_ego=h_ego,
                wn_Ws=[p['wn%d_1W' % t] for t in range(3)],
                wn_bs=[p['wn%d_1b' % t].reshape(1, H) for t in range(3)])

    emlp = _sc_edge_mlp(nc1, k1)
    segv = _sc_segadd_vals(nc1, k1)
    weights = []
    qs = []
    for t in range(3):
        dst_g = _pad_idx(eis[t][1], epad1, 0)
        dst_s = _pad_idx(eis[t][1], epad1, N)
        src = _pad_idx(eis[t][0], epad1, 0)
        w2m = jnp.broadcast_to(p['wn%d_2W' % t], (H, 16))
        b2v = jnp.broadcast_to(p['wn%d_2b' % t], (16,))
        u_t, w_t = emlp(ab[2 * t], ab[2 * t + 1], treat16, dst_g, src,
                        w2m, b2v)
        weights.append(w_t[:E])
        qs.append(segv(u_t, dst_s))

    y_full, y_local, exposure, gps_mu, gps_logvar = _tc_heads(
        h_ego, h_gnn, treat, qs, p)

    return (y_full, y_local, exposure, tuple(weights), gps_mu, gps_logvar)
